# baseline (speedup 1.0000x reference)
import functools

import jax
import jax.numpy as jnp
from jax import lax
from jax.experimental import pallas as pl
from jax.experimental.pallas import tpu as pltpu
from jax.experimental.pallas import tpu_sc as plsc

DIM = 64
NC, NS = 2, 16
NW = NC * NS


def _sc_gather(table, center_idx, context_idx):
    batch = center_idx.shape[0]
    bpw = batch // NW
    mesh = plsc.VectorSubcoreMesh(core_axis_name="c", subcore_axis_name="s")

    @functools.partial(
        pl.kernel,
        mesh=mesh,
        out_type=[jax.ShapeDtypeStruct((batch, DIM), jnp.float32),
                  jax.ShapeDtypeStruct((batch, DIM), jnp.float32)],
        scratch_types=[
            pltpu.VMEM((bpw,), jnp.int32),
            pltpu.VMEM((bpw,), jnp.int32),
            pltpu.SemaphoreType.DMA,
            pltpu.SemaphoreType.DMA,
        ],
    )
    def k(table_hbm, cen_hbm, ctx_hbm, out_cen_hbm, out_ctx_hbm,
          cen_idx_v, ctx_idx_v, sem_c, sem_x):
        wid = lax.axis_index("s") * NC + lax.axis_index("c")
        base = wid * bpw
        pltpu.sync_copy(cen_hbm.at[pl.ds(base, bpw)], cen_idx_v)
        pltpu.sync_copy(ctx_hbm.at[pl.ds(base, bpw)], ctx_idx_v)

        @pl.loop(0, bpw, step=16)
        def _(g):
            cvec = cen_idx_v[pl.ds(g, 16)]
            xvec = ctx_idx_v[pl.ds(g, 16)]
            for j in range(16):
                pltpu.async_copy(table_hbm.at[pl.ds(cvec[j], 1)],
                                 out_cen_hbm.at[pl.ds(base + g + j, 1)], sem_c)
                pltpu.async_copy(table_hbm.at[pl.ds(xvec[j], 1)],
                                 out_ctx_hbm.at[pl.ds(base + g + j, 1)], sem_x)

        pltpu.make_async_copy(table_hbm.at[pl.ds(0, bpw)],
                              out_cen_hbm.at[pl.ds(base, bpw)], sem_c).wait()
        pltpu.make_async_copy(table_hbm.at[pl.ds(0, bpw)],
                              out_ctx_hbm.at[pl.ds(base, bpw)], sem_x).wait()

    return k(table, center_idx, context_idx)


def _tc_cosine_body(c_ref, x_ref, o_ref):
    c = c_ref[...]
    x = x_ref[...]
    dot = jnp.sum(c * x, axis=1)
    cn = jnp.sqrt(jnp.sum(c * c, axis=1))
    xn = jnp.sqrt(jnp.sum(x * x, axis=1))
    o_ref[...] = dot / (cn * xn + 1e-8)


def _tc_cosine(center_embed, context_embed):
    batch = center_embed.shape[0]
    return pl.pallas_call(
        _tc_cosine_body,
        out_shape=jax.ShapeDtypeStruct((batch,), jnp.float32),
    )(center_embed, context_embed)


@jax.jit
def kernel(center_idx, context_idx, table):
    ce, xe = _sc_gather(table,
                        center_idx.astype(jnp.int32),
                        context_idx.astype(jnp.int32))
    return _tc_cosine(ce, xe)


# speedup vs baseline: 1.3972x; 1.3972x over previous
import functools

import jax
import jax.numpy as jnp
from jax import lax
from jax.experimental import pallas as pl
from jax.experimental.pallas import tpu as pltpu
from jax.experimental.pallas import tpu_sc as plsc

DIM = 64
NC, NS = 2, 16
NW = NC * NS
CHUNK = 256
GRP = 16


def _sc_cosine(table, center_idx, context_idx):
    batch = center_idx.shape[0]
    bpw = batch // NW
    nchunks = bpw // CHUNK
    mesh = plsc.VectorSubcoreMesh(core_axis_name="c", subcore_axis_name="s")

    @functools.partial(
        pl.kernel,
        mesh=mesh,
        compiler_params=pltpu.CompilerParams(use_tc_tiling_on_sc=False,
                                             needs_layout_passes=False),
        out_type=jax.ShapeDtypeStruct((batch,), jnp.float32),
        scratch_types=[
            pltpu.VMEM((bpw,), jnp.int32),
            pltpu.VMEM((bpw,), jnp.int32),
            pltpu.VMEM((CHUNK, DIM), jnp.float32),
            pltpu.VMEM((CHUNK, DIM), jnp.float32),
            pltpu.VMEM((bpw,), jnp.float32),
            pltpu.SemaphoreType.DMA,
            pltpu.SemaphoreType.DMA,
        ],
    )
    def k(table_hbm, cen_hbm, ctx_hbm, out_hbm,
          rcen_v, rctx_v, dstc_v, dstx_v, out_v, sem_c, sem_x):
        wid = lax.axis_index("s") * NC + lax.axis_index("c")
        base = wid * bpw
        pltpu.sync_copy(cen_hbm.at[pl.ds(base, bpw)], rcen_v)
        pltpu.sync_copy(ctx_hbm.at[pl.ds(base, bpw)], rctx_v)

        for chunk in range(nchunks):
            cbase = chunk * CHUNK
            cp_c = pltpu.async_copy(
                table_hbm.at[rcen_v.at[pl.ds(cbase, CHUNK)]], dstc_v, sem_c)
            cp_x = pltpu.async_copy(
                table_hbm.at[rctx_v.at[pl.ds(cbase, CHUNK)]], dstx_v, sem_x)
            cp_c.wait()
            cp_x.wait()

            @pl.loop(0, CHUNK // GRP)
            def _compute(g):
                lane = lax.iota(jnp.int32, GRP)
                rows = jnp.full((GRP,), 0, jnp.int32) + g * GRP + lane
                dot = jnp.zeros((GRP,), jnp.float32)
                cc = jnp.zeros((GRP,), jnp.float32)
                xx = jnp.zeros((GRP,), jnp.float32)
                for c in range(DIM):
                    cols = (jnp.full((GRP,), c, jnp.int32) + lane) & (DIM - 1)
                    cv = plsc.load_gather(dstc_v, [rows, cols])
                    xv = plsc.load_gather(dstx_v, [rows, cols])
                    dot = dot + cv * xv
                    cc = cc + cv * cv
                    xx = xx + xv * xv
                y = cc * xx
                iy = plsc.bitcast(y, jnp.int32)
                iz = jnp.int32(0x5F3759DF) - lax.shift_right_logical(iy, 1)
                z = plsc.bitcast(iz, jnp.float32)
                for _ in range(3):
                    z = z * (1.5 - 0.5 * y * z * z)
                denom = y * z
                out_v[pl.ds(cbase + g * GRP, GRP)] = dot / (denom + 1e-8)

        pltpu.sync_copy(out_v, out_hbm.at[pl.ds(base, bpw)])

    return k(table, center_idx, context_idx)


@jax.jit
def kernel(center_idx, context_idx, table):
    return _sc_cosine(table,
                      center_idx.astype(jnp.int32),
                      context_idx.astype(jnp.int32))


# speedup vs baseline: 1.3983x; 1.0008x over previous
import functools

import jax
import jax.numpy as jnp
from jax import lax
from jax.experimental import pallas as pl
from jax.experimental.pallas import tpu as pltpu
from jax.experimental.pallas import tpu_sc as plsc

DIM = 64
NC, NS = 2, 16
NW = NC * NS
CHUNK = 256
GRP = 16


def _sc_cosine(table2, center_idx, context_idx):
    batch = center_idx.shape[0]
    bpw = batch // NW
    nchunks = bpw // CHUNK
    mesh = plsc.VectorSubcoreMesh(core_axis_name="c", subcore_axis_name="s")

    @functools.partial(
        pl.kernel,
        mesh=mesh,
        compiler_params=pltpu.CompilerParams(needs_layout_passes=False),
        out_type=jax.ShapeDtypeStruct((batch,), jnp.float32),
        scratch_types=[
            pltpu.VMEM((bpw,), jnp.int32),
            pltpu.VMEM((bpw,), jnp.int32),
            pltpu.VMEM((bpw,), jnp.int32),
            pltpu.VMEM((bpw,), jnp.int32),
            pltpu.VMEM((CHUNK, 2 * DIM), jnp.float32),
            pltpu.VMEM((CHUNK, 2 * DIM), jnp.float32),
            pltpu.VMEM((bpw,), jnp.float32),
            pltpu.SemaphoreType.DMA,
            pltpu.SemaphoreType.DMA,
        ],
    )
    def k(table_hbm, cen_hbm, ctx_hbm, out_hbm,
          rcen_v, rctx_v, pcen_v, pctx_v, dstc_v, dstx_v, out_v,
          sem_c, sem_x):
        wid = lax.axis_index("s") * NC + lax.axis_index("c")
        base = wid * bpw
        pltpu.sync_copy(cen_hbm.at[pl.ds(base, bpw)], rcen_v)
        pltpu.sync_copy(ctx_hbm.at[pl.ds(base, bpw)], rctx_v)

        @pl.loop(0, bpw // GRP)
        def _pairs(g):
            sl = pl.ds(g * GRP, GRP)
            pcen_v[sl] = lax.shift_right_logical(rcen_v[sl], 1)
            pctx_v[sl] = lax.shift_right_logical(rctx_v[sl], 1)

        for chunk in range(nchunks):
            cbase = chunk * CHUNK
            cp_c = pltpu.async_copy(
                table_hbm.at[pcen_v.at[pl.ds(cbase, CHUNK)]], dstc_v, sem_c)
            cp_x = pltpu.async_copy(
                table_hbm.at[pctx_v.at[pl.ds(cbase, CHUNK)]], dstx_v, sem_x)
            cp_c.wait()
            cp_x.wait()

            @pl.loop(0, CHUNK // GRP)
            def _compute(g):
                lane = lax.iota(jnp.int32, GRP)
                rows = g * GRP + lane
                sl = pl.ds(cbase + g * GRP, GRP)
                pc = (rcen_v[sl] & 1) * DIM
                px = (rctx_v[sl] & 1) * DIM
                dot = jnp.zeros((GRP,), jnp.float32)
                cc = jnp.zeros((GRP,), jnp.float32)
                xx = jnp.zeros((GRP,), jnp.float32)
                for c in range(DIM):
                    rot = (jnp.full((GRP,), c, jnp.int32) + lane) & (DIM - 1)
                    cv = plsc.load_gather(dstc_v, [rows, pc + rot])
                    xv = plsc.load_gather(dstx_v, [rows, px + rot])
                    dot = dot + cv * xv
                    cc = cc + cv * cv
                    xx = xx + xv * xv
                y = cc * xx
                iy = plsc.bitcast(y, jnp.int32)
                iz = jnp.int32(0x5F3759DF) - lax.shift_right_logical(iy, 1)
                z = plsc.bitcast(iz, jnp.float32)
                for _ in range(3):
                    z = z * (1.5 - 0.5 * y * z * z)
                denom = y * z
                out_v[pl.ds(cbase + g * GRP, GRP)] = dot / (denom + 1e-8)

        pltpu.sync_copy(out_v, out_hbm.at[pl.ds(base, bpw)])

    return k(table2, center_idx, context_idx)


@jax.jit
def kernel(center_idx, context_idx, table):
    table2 = jnp.reshape(table, (table.shape[0] // 2, 2 * DIM))
    return _sc_cosine(table2,
                      center_idx.astype(jnp.int32),
                      context_idx.astype(jnp.int32))
